# baseline (speedup 1.0000x reference)
import jax
import jax.numpy as jnp
from jax.experimental import pallas as pl
from jax.experimental.pallas import tpu as pltpu

L = 512
NSEQ = 64
D_MSA = 64
D_NODE = 32
D_EDGE = 32
D_HID = 64
KMIN = 9

TI = 128
TJ = 128

BF = jnp.bfloat16
F32 = jnp.float32


def _ln(x, eps=1e-5):
    m = jnp.mean(x, axis=-1, keepdims=True)
    v = jnp.mean((x - m) ** 2, axis=-1, keepdims=True)
    return (x - m) / jnp.sqrt(v + eps)


def _b(x):
    return x.astype(BF)


def _bf(x):
    return x.astype(BF).astype(F32)


def _dot(a, b):
    return jnp.dot(a, b, preferred_element_type=F32)


def _frontend_kernel(msa_ref, seq_ref, wq_ref, wk_ref, wxm_ref, wxs_ref,
                     wms_ref, wmt_ref, bm_ref, hsrc_ref, htgt_ref):
    msa = msa_ref[0]
    msa_n = _ln(msa)
    msa_nb = _b(msa_n)
    q = _dot(msa_nb[0], wq_ref[...]) * F32(0.125)
    km = _dot(msa_nb.reshape(NSEQ * msa.shape[1], D_MSA),
              wk_ref[...]).reshape(msa.shape[0], msa.shape[1], D_MSA)
    logits = jnp.sum(_bf(q)[None, :, :] * _bf(km), axis=-1)
    mx = jnp.max(logits, axis=0, keepdims=True)
    e = jnp.exp(logits - mx)
    attnw = e / jnp.sum(e, axis=0, keepdims=True)
    msum = jnp.sum(attnw[:, :, None] * msa_n, axis=0)
    node_pre = _dot(_b(msum), wxm_ref[...]) + _dot(_b(seq_ref[0]), wxs_ref[...])
    node = _ln(node_pre)
    node_b = _b(node)
    hsrc_ref[...] = _dot(node_b, wms_ref[...]) + bm_ref[...]
    htgt_ref[...] = _dot(node_b, wmt_ref[...])


def _mask_kernel(cax_ref, cay_ref, caz_ref, idx_ref, topk_ref, cond_ref):
    dx = cax_ref[0][:, None] - cax_ref[0][None, :]
    dy = cay_ref[0][:, None] - cay_ref[0][None, :]
    dz = caz_ref[0][:, None] - caz_ref[0][None, :]
    d2 = dx * dx + dy * dy + dz * dz
    eye = (jax.lax.broadcasted_iota(jnp.int32, (L, L), 0)
           == jax.lax.broadcasted_iota(jnp.int32, (L, L), 1))
    D = jnp.sqrt(d2 + 1e-12) + jnp.where(eye, 999.9, 0.0).astype(F32)
    bits = jax.lax.bitcast_convert_type(D, jnp.int32)
    kk = jnp.minimum(topk_ref[...], L)

    def body(_, carry):
        lo, hi = carry
        mid = lo + (hi - lo) // 2
        cnt = jnp.sum((bits <= mid).astype(jnp.int32), axis=-1, keepdims=True)
        ge = cnt >= kk
        return jnp.where(ge, lo, mid), jnp.where(ge, mid, hi)

    lo0 = jnp.zeros((L, 1), jnp.int32)
    hi0 = jnp.full((L, 1), jnp.int32(0x7F7FFFFF))
    _, hi = jax.lax.fori_loop(0, 31, body, (lo0, hi0))
    topk_mask = bits <= hi

    idx = idx_ref[0]
    sep = jnp.abs(idx[:, None] - idx[None, :])
    band = jnp.logical_and(sep < KMIN, jnp.logical_not(eye))
    cond_ref[...] = jnp.logical_or(topk_mask, band).astype(F32)


def _main_kernel(pair_ref, cond_ref, hsrc_ref, htgt_ref, cai_ref, caj_ref,
                 we_ref, wme_ref, wmd_ref, wv_ref, ws_ref,
                 state_ref, xyz_ref, agg_acc, aggv_acc, u_acc):
    ni = pl.num_programs(1)
    i = pl.program_id(1)

    p = pair_ref[0].reshape(TI * TJ, 128)
    pn = _ln(p)
    edge = _ln(_dot(_b(pn), we_ref[...]))
    msg_e = _dot(_b(edge), wme_ref[...]).reshape(TI, TJ, D_HID)

    dist2 = jnp.zeros((TI, TJ), F32)
    for x in range(3):
        diff = caj_ref[:, x][None, :] - cai_ref[:, x][:, None]
        dist2 = dist2 + diff * diff
    dist_b = _bf(jnp.sqrt(dist2))

    mmsg = jax.nn.relu(msg_e
                       + hsrc_ref[...][:, None, :]
                       + htgt_ref[...][None, :, :]
                       + dist_b[:, :, None] * wmd_ref[...][None, :, :])
    cnd = cond_ref[...][:, :, None]
    masked = mmsg * cnd
    maskedv = _bf(mmsg) * cnd

    agg_t = jnp.sum(masked, axis=0)
    aggv_t = jnp.sum(maskedv, axis=0)
    u_t = jnp.stack(
        [jnp.sum(maskedv * cai_ref[:, x][:, None, None], axis=0)
         for x in range(3)], axis=0)

    @pl.when(i == 0)
    def _init():
        agg_acc[...] = agg_t
        aggv_acc[...] = aggv_t
        u_acc[...] = u_t

    @pl.when(i > 0)
    def _accum():
        agg_acc[...] += agg_t
        aggv_acc[...] += aggv_t
        u_acc[...] += u_t

    @pl.when(i == ni - 1)
    def _epilogue():
        state_ref[...] = _dot(_b(agg_acc[...]), ws_ref[...])
        wv = wv_ref[...]
        aggv = aggv_acc[...]
        s = [jnp.sum(aggv * wv[:, c][None, :], axis=1) for c in range(3)]
        for x in range(3):
            u_x = u_acc[x]
            t = [jnp.sum(u_x * wv[:, c][None, :], axis=1) for c in range(3)]
            cax = caj_ref[:, x]
            off = [cax * s[c] - t[c] for c in range(3)]
            ca_new = cax + off[1]
            xyz_ref[:, 0 + x] = ca_new + off[0]
            xyz_ref[:, 3 + x] = ca_new
            xyz_ref[:, 6 + x] = ca_new + off[2]


def kernel(msa, pair, xyz, seq1hot, idx, top_k, Wq, Wk, Wx, We, Wm, bm, Ws, Wv):
    msa = msa.astype(F32)
    ca = xyz[:, :, 1, :].astype(F32)
    ca2 = ca[0]
    cax = ca2[:, 0][None, :]
    cay = ca2[:, 1][None, :]
    caz = ca2[:, 2][None, :]
    topk_arr = jnp.asarray(top_k, jnp.int32).reshape(1, 1)

    TL = 128
    hsrc, htgt = pl.pallas_call(
        _frontend_kernel,
        grid=(L // TL,),
        in_specs=[
            pl.BlockSpec((1, NSEQ, TL, D_MSA), lambda l: (0, 0, l, 0)),
            pl.BlockSpec((1, TL, 21), lambda l: (0, l, 0)),
            pl.BlockSpec((D_MSA, D_MSA), lambda l: (0, 0)),
            pl.BlockSpec((D_MSA, D_MSA), lambda l: (0, 0)),
            pl.BlockSpec((D_MSA, D_NODE), lambda l: (0, 0)),
            pl.BlockSpec((21, D_NODE), lambda l: (0, 0)),
            pl.BlockSpec((D_NODE, D_HID), lambda l: (0, 0)),
            pl.BlockSpec((D_NODE, D_HID), lambda l: (0, 0)),
            pl.BlockSpec((1, D_HID), lambda l: (0, 0)),
        ],
        out_specs=[
            pl.BlockSpec((TL, D_HID), lambda l: (l, 0)),
            pl.BlockSpec((TL, D_HID), lambda l: (l, 0)),
        ],
        out_shape=[jax.ShapeDtypeStruct((L, D_HID), F32),
                   jax.ShapeDtypeStruct((L, D_HID), F32)],
    )(msa, seq1hot, _b(Wq), _b(Wk), _b(Wx[:D_MSA]), _b(Wx[D_MSA:]),
      _b(Wm[:D_NODE]), _b(Wm[D_NODE:2 * D_NODE]), bm[None])

    cond = pl.pallas_call(
        _mask_kernel,
        out_shape=jax.ShapeDtypeStruct((L, L), F32),
    )(cax, cay, caz, idx, topk_arr)

    nj, ni = L // TJ, L // TI
    state, xyz9 = pl.pallas_call(
        _main_kernel,
        grid=(nj, ni),
        in_specs=[
            pl.BlockSpec((1, TI, TJ, 128), lambda j, i: (0, i, j, 0)),
            pl.BlockSpec((TI, TJ), lambda j, i: (i, j)),
            pl.BlockSpec((TI, D_HID), lambda j, i: (i, 0)),
            pl.BlockSpec((TJ, D_HID), lambda j, i: (j, 0)),
            pl.BlockSpec((TI, 3), lambda j, i: (i, 0)),
            pl.BlockSpec((TJ, 3), lambda j, i: (j, 0)),
            pl.BlockSpec((128, D_EDGE), lambda j, i: (0, 0)),
            pl.BlockSpec((D_EDGE, D_HID), lambda j, i: (0, 0)),
            pl.BlockSpec((1, D_HID), lambda j, i: (0, 0)),
            pl.BlockSpec((D_HID, 3), lambda j, i: (0, 0)),
            pl.BlockSpec((D_HID, 16), lambda j, i: (0, 0)),
        ],
        out_specs=[
            pl.BlockSpec((TJ, 16), lambda j, i: (j, 0)),
            pl.BlockSpec((TJ, 9), lambda j, i: (j, 0)),
        ],
        out_shape=[jax.ShapeDtypeStruct((L, 16), F32),
                   jax.ShapeDtypeStruct((L, 9), F32)],
        scratch_shapes=[pltpu.VMEM((TJ, D_HID), F32),
                        pltpu.VMEM((TJ, D_HID), F32),
                        pltpu.VMEM((3, TJ, D_HID), F32)],
        compiler_params=pltpu.CompilerParams(
            dimension_semantics=("arbitrary", "arbitrary")),
    )(pair, cond, hsrc, htgt, ca2, ca2, _b(We),
      _b(Wm[2 * D_NODE:3 * D_NODE]), _bf(Wm[3 * D_NODE])[None],
      _bf(Wv), _b(Ws))

    xyz_new = xyz9.reshape(1, L, 3, 3)
    return xyz_new, state[None]


# speedup vs baseline: 1.3306x; 1.3306x over previous
import jax
import jax.numpy as jnp
from jax.experimental import pallas as pl
from jax.experimental.pallas import tpu as pltpu

L = 512
NSEQ = 64
D_MSA = 64
D_NODE = 32
D_EDGE = 32
D_HID = 64
KMIN = 9

TI = 128
TJ = 128

BF = jnp.bfloat16
F32 = jnp.float32


def _ln(x, eps=1e-5):
    m = jnp.mean(x, axis=-1, keepdims=True)
    v = jnp.mean((x - m) ** 2, axis=-1, keepdims=True)
    return (x - m) / jnp.sqrt(v + eps)


def _b(x):
    return x.astype(BF)


def _bf(x):
    return x.astype(BF).astype(F32)


def _dot(a, b):
    return jnp.dot(a, b, preferred_element_type=F32)


def _ln_fast(x):
    n = x.shape[-1]
    ones_col = jnp.ones((n, 1), BF)
    m = jnp.dot(_b(x), ones_col, preferred_element_type=F32) * F32(1.0 / n)
    t = x - m
    v = jnp.dot(_b(t * t), ones_col, preferred_element_type=F32) * F32(1.0 / n)
    return t * (F32(1.0) / jnp.sqrt(v + 1e-5))


def _frontend_kernel(msa_ref, seq_ref, wq_ref, wk_ref, wxm_ref, wxs_ref,
                     wms_ref, wmt_ref, bm_ref, hsrc_ref, htgt_ref):
    msa = msa_ref[0]
    msa_n = _ln(msa)
    msa_nb = _b(msa_n)
    q = _dot(msa_nb[0], wq_ref[...]) * F32(0.125)
    km = _dot(msa_nb.reshape(NSEQ * msa.shape[1], D_MSA),
              wk_ref[...]).reshape(msa.shape[0], msa.shape[1], D_MSA)
    logits = jnp.sum(_bf(q)[None, :, :] * _bf(km), axis=-1)
    mx = jnp.max(logits, axis=0, keepdims=True)
    e = jnp.exp(logits - mx)
    attnw = e / jnp.sum(e, axis=0, keepdims=True)
    msum = jnp.sum(attnw[:, :, None] * msa_n, axis=0)
    node_pre = _dot(_b(msum), wxm_ref[...]) + _dot(_b(seq_ref[0]), wxs_ref[...])
    node = _ln(node_pre)
    node_b = _b(node)
    hsrc_ref[...] = _dot(node_b, wms_ref[...]) + bm_ref[...]
    htgt_ref[...] = _dot(node_b, wmt_ref[...])


def _mask_kernel(cax_ref, cay_ref, caz_ref, idx_ref, topk_ref,
                 cond_ref, dist_ref):
    dx = cax_ref[0][:, None] - cax_ref[0][None, :]
    dy = cay_ref[0][:, None] - cay_ref[0][None, :]
    dz = caz_ref[0][:, None] - caz_ref[0][None, :]
    d2 = dx * dx + dy * dy + dz * dz
    dist_ref[...] = jnp.sqrt(d2).astype(BF)
    eye = (jax.lax.broadcasted_iota(jnp.int32, (L, L), 0)
           == jax.lax.broadcasted_iota(jnp.int32, (L, L), 1))
    D = jnp.sqrt(d2 + 1e-12) + jnp.where(eye, 999.9, 0.0).astype(F32)
    bits = jax.lax.bitcast_convert_type(D, jnp.int32)
    kk = jnp.minimum(topk_ref[...], L)

    def body(_, carry):
        lo, hi = carry
        mid = lo + (hi - lo) // 2
        cnt = jnp.sum((bits <= mid).astype(jnp.int32), axis=-1, keepdims=True)
        ge = cnt >= kk
        return jnp.where(ge, lo, mid), jnp.where(ge, mid, hi)

    lo0 = jnp.zeros((L, 1), jnp.int32)
    hi0 = jnp.full((L, 1), jnp.int32(0x7F7FFFFF))
    _, hi = jax.lax.fori_loop(0, 31, body, (lo0, hi0))
    topk_mask = bits <= hi

    idx = idx_ref[0]
    sep = jnp.abs(idx[:, None] - idx[None, :])
    band = jnp.logical_and(sep < KMIN, jnp.logical_not(eye))
    cond_ref[...] = jnp.logical_or(topk_mask, band).astype(F32)


def _main_kernel(pair_ref, cond_ref, dist_ref, hsrc_ref, htgt_ref, cai_ref,
                 we_ref, wme_ref, wmd_ref, acc_ref):
    i = pl.program_id(1)

    p = pair_ref[0].reshape(TI * TJ, 128)
    pn = _ln_fast(p)
    edge = _ln_fast(_dot(_b(pn), we_ref[...]))
    msg_e = _dot(_b(edge), wme_ref[...]).reshape(TI, TJ, D_HID)

    dist_b = dist_ref[...].astype(F32)

    mmsg = jax.nn.relu(msg_e
                       + hsrc_ref[...][:, None, :]
                       + htgt_ref[...][None, :, :]
                       + dist_b[:, :, None] * wmd_ref[...][None, :, :])
    mb = (mmsg * cond_ref[...][:, :, None]).astype(BF)
    mb2 = mb.reshape(TI, TJ * D_HID)

    cai = cai_ref[...]
    chi = cai.astype(BF)
    clo = (cai - chi.astype(F32)).astype(BF)
    one = jnp.ones((TI,), BF)
    lhs = jnp.stack([one, chi[:, 0], clo[:, 0], chi[:, 1], clo[:, 1],
                     chi[:, 2], clo[:, 2], jnp.zeros((TI,), BF)], axis=0)
    part = jax.lax.dot_general(lhs, mb2, (((1,), (0,)), ((), ())),
                               preferred_element_type=F32)

    @pl.when(i == 0)
    def _init():
        acc_ref[...] = part

    @pl.when(i > 0)
    def _accum():
        acc_ref[...] += part


def _epilogue_kernel(acc_ref, ca_ref, wv_ref, ws_ref, state_ref, xyz_ref):
    aggv = acc_ref[0]
    state_ref[...] = _dot(_b(aggv), ws_ref[...])
    wv = wv_ref[...]
    s = [jnp.sum(aggv * wv[:, c][None, :], axis=1) for c in range(3)]
    for x in range(3):
        u_x = acc_ref[2 * x + 1] + acc_ref[2 * x + 2]
        t = [jnp.sum(u_x * wv[:, c][None, :], axis=1) for c in range(3)]
        cax = ca_ref[:, x]
        off = [cax * s[c] - t[c] for c in range(3)]
        ca_new = cax + off[1]
        xyz_ref[:, 0 + x] = ca_new + off[0]
        xyz_ref[:, 3 + x] = ca_new
        xyz_ref[:, 6 + x] = ca_new + off[2]


def kernel(msa, pair, xyz, seq1hot, idx, top_k, Wq, Wk, Wx, We, Wm, bm, Ws, Wv):
    msa = msa.astype(F32)
    ca = xyz[:, :, 1, :].astype(F32)
    ca2 = ca[0]
    cax = ca2[:, 0][None, :]
    cay = ca2[:, 1][None, :]
    caz = ca2[:, 2][None, :]
    topk_arr = jnp.asarray(top_k, jnp.int32).reshape(1, 1)

    TL = 128
    hsrc, htgt = pl.pallas_call(
        _frontend_kernel,
        grid=(L // TL,),
        in_specs=[
            pl.BlockSpec((1, NSEQ, TL, D_MSA), lambda l: (0, 0, l, 0)),
            pl.BlockSpec((1, TL, 21), lambda l: (0, l, 0)),
            pl.BlockSpec((D_MSA, D_MSA), lambda l: (0, 0)),
            pl.BlockSpec((D_MSA, D_MSA), lambda l: (0, 0)),
            pl.BlockSpec((D_MSA, D_NODE), lambda l: (0, 0)),
            pl.BlockSpec((21, D_NODE), lambda l: (0, 0)),
            pl.BlockSpec((D_NODE, D_HID), lambda l: (0, 0)),
            pl.BlockSpec((D_NODE, D_HID), lambda l: (0, 0)),
            pl.BlockSpec((1, D_HID), lambda l: (0, 0)),
        ],
        out_specs=[
            pl.BlockSpec((TL, D_HID), lambda l: (l, 0)),
            pl.BlockSpec((TL, D_HID), lambda l: (l, 0)),
        ],
        out_shape=[jax.ShapeDtypeStruct((L, D_HID), F32),
                   jax.ShapeDtypeStruct((L, D_HID), F32)],
    )(msa, seq1hot, _b(Wq), _b(Wk), _b(Wx[:D_MSA]), _b(Wx[D_MSA:]),
      _b(Wm[:D_NODE]), _b(Wm[D_NODE:2 * D_NODE]), bm[None])

    cond, distb = pl.pallas_call(
        _mask_kernel,
        out_shape=[jax.ShapeDtypeStruct((L, L), F32),
                   jax.ShapeDtypeStruct((L, L), BF)],
    )(cax, cay, caz, idx, topk_arr)

    nj, ni = L // TJ, L // TI
    acc = pl.pallas_call(
        _main_kernel,
        grid=(nj, ni),
        in_specs=[
            pl.BlockSpec((1, TI, TJ, 128), lambda j, i: (0, i, j, 0)),
            pl.BlockSpec((TI, TJ), lambda j, i: (i, j)),
            pl.BlockSpec((TI, TJ), lambda j, i: (i, j)),
            pl.BlockSpec((TI, D_HID), lambda j, i: (i, 0)),
            pl.BlockSpec((TJ, D_HID), lambda j, i: (j, 0)),
            pl.BlockSpec((TI, 3), lambda j, i: (i, 0)),
            pl.BlockSpec((128, D_EDGE), lambda j, i: (0, 0)),
            pl.BlockSpec((D_EDGE, D_HID), lambda j, i: (0, 0)),
            pl.BlockSpec((1, D_HID), lambda j, i: (0, 0)),
        ],
        out_specs=pl.BlockSpec((8, TJ * D_HID), lambda j, i: (0, j)),
        out_shape=jax.ShapeDtypeStruct((8, L * D_HID), F32),
        compiler_params=pltpu.CompilerParams(
            dimension_semantics=("arbitrary", "arbitrary")),
    )(pair, cond, distb, hsrc, htgt, ca2, _b(We),
      _b(Wm[2 * D_NODE:3 * D_NODE]), _bf(Wm[3 * D_NODE])[None])

    state, xyz9 = pl.pallas_call(
        _epilogue_kernel,
        out_shape=[jax.ShapeDtypeStruct((L, 16), F32),
                   jax.ShapeDtypeStruct((L, 9), F32)],
    )(acc.reshape(8, L, D_HID), ca2, _bf(Wv), _b(Ws))

    xyz_new = xyz9.reshape(1, L, 3, 3)
    return xyz_new, state[None]
